# baseline (speedup 1.0000x reference)
import functools

import jax
import jax.numpy as jnp
from jax import lax
from jax.experimental import pallas as pl
from jax.experimental.pallas import tpu as pltpu
from jax.experimental.pallas import tpu_sc as plsc

N = 10000
E = 160000
D = 256
L = 4
G = 64
ATOM_DIMS = [119, 4, 12, 12, 10, 6, 6, 2, 2]

NPAD = 10240
NB = 128
P = NPAD // NB
EPAD = E + 256
C = 96
NC = 2
NS = 16

_mesh = plsc.VectorSubcoreMesh(core_axis_name="c", subcore_axis_name="s")


def _edge_body(h_in, srcs, pks, bnds, comb,
               mx, mn, sm,
               bounds_v, combo_v, src_v0, src_v1, pk_v0, pk_v1,
               rows_v0, rows_v1, amx, amn, asm_,
               ssem0, ssem1, psem0, psem1, gsem0, gsem1):
    cidx = lax.axis_index("c")
    sidx = lax.axis_index("s")
    w = sidx * NC + cidx
    srcv = (src_v0, src_v1)
    pkv = (pk_v0, pk_v1)
    rowsv = (rows_v0, rows_v1)
    ssem = (ssem0, ssem1)
    psem = (psem0, psem1)
    gsem = (gsem0, gsem1)

    pltpu.sync_copy(bnds, bounds_v)
    pltpu.sync_copy(comb, combo_v)

    neg = jnp.full((16,), -jnp.inf, jnp.float32)
    pos = jnp.full((16,), jnp.inf, jnp.float32)
    zer = jnp.zeros((16,), jnp.float32)

    def bucket(bk, carry):
        b = w + 32 * bk
        node_base = b * P
        start = bounds_v[pl.ds(b, 16)][0]
        end = bounds_v[pl.ds(b + 1, 16)][0]
        start_a = start & jnp.int32(-8)
        nch = (end - start_a + (C - 1)) // C

        def initrow(r, cr):
            for jj in range(16):
                sl = pl.ds(16 * jj, 16)
                amx[r, sl] = neg
                amn[r, sl] = pos
                asm_[r, sl] = zer
            return cr
        lax.fori_loop(0, P, initrow, 0)

        def fire_meta(ci, k):
            base = pl.multiple_of(start_a + ci * C, 8)
            pltpu.async_copy(srcs.at[pl.ds(base, C)], srcv[k], ssem[k])
            pltpu.async_copy(pks.at[pl.ds(base, C)],
                             pkv[k].at[pl.ds(0, C)], psem[k])

        def wait_meta(k):
            pltpu.make_async_copy(srcs.at[pl.ds(0, C)], srcv[k],
                                  ssem[k]).wait()
            pltpu.make_async_copy(pks.at[pl.ds(0, C)],
                                  pkv[k].at[pl.ds(0, C)], psem[k]).wait()

        def fire_gather(k):
            pltpu.async_copy(h_in.at[srcv[k]], rowsv[k], gsem[k])

        def wait_gather(k):
            pltpu.make_async_copy(h_in.at[srcv[k]], rowsv[k], gsem[k]).wait()

        @pl.when(nch > 0)
        def _():
            fire_meta(0, 0)

            @pl.when(nch > 1)
            def _():
                fire_meta(1, 1)
            wait_meta(0)
            fire_gather(0)

        def step(ci, k):
            wait_gather(k)

            @pl.when(ci + 1 < nch)
            def _():
                wait_meta(1 - k)
                fire_gather(1 - k)

            def group(g, ec, k=k):
                gv = pkv[k][pl.ds(g * 16, 16)]
                for u in range(16):
                    v = gv[u]
                    off = lax.shift_right_logical(v, 6) - node_base
                    ck = v & 63
                    e = g * 16 + u

                    @pl.when((off >= 0) & (off < P))
                    def _(e=e, off=off, ck=ck, k=k):
                        for jj in range(16):
                            sl = pl.ds(16 * jj, 16)
                            m = rowsv[k][e, sl] + combo_v[ck, sl]
                            amx[off, sl] = jnp.maximum(amx[off, sl], m)
                            amn[off, sl] = jnp.minimum(amn[off, sl], m)
                            asm_[off, sl] = asm_[off, sl] + m
                return ec
            lax.fori_loop(0, C // 16, group, 0)

            @pl.when(ci + 2 < nch)
            def _():
                fire_meta(ci + 2, k)

        def pair(i, cr):
            ci0 = 2 * i
            step(ci0, 0)

            @pl.when(ci0 + 1 < nch)
            def _():
                step(ci0 + 1, 1)
            return cr
        lax.fori_loop(0, (nch + 1) // 2, pair, 0)

        pltpu.sync_copy(amx, mx.at[pl.ds(node_base, P)])
        pltpu.sync_copy(amn, mn.at[pl.ds(node_base, P)])
        pltpu.sync_copy(asm_, sm.at[pl.ds(node_base, P)])
        return carry
    lax.fori_loop(0, 4, bucket, 0)


_edge_kernel = functools.partial(
    pl.kernel,
    out_type=[jax.ShapeDtypeStruct((NPAD, D), jnp.float32),
              jax.ShapeDtypeStruct((NPAD, D), jnp.float32),
              jax.ShapeDtypeStruct((NPAD, D), jnp.float32)],
    mesh=_mesh,
    scratch_types=[
        pltpu.VMEM((144,), jnp.int32),
        pltpu.VMEM((64, D), jnp.float32),
        pltpu.VMEM((C,), jnp.int32),
        pltpu.VMEM((C,), jnp.int32),
        pltpu.VMEM((C + 16,), jnp.int32),
        pltpu.VMEM((C + 16,), jnp.int32),
        pltpu.VMEM((C, D), jnp.float32),
        pltpu.VMEM((C, D), jnp.float32),
        pltpu.VMEM((P, D), jnp.float32),
        pltpu.VMEM((P, D), jnp.float32),
        pltpu.VMEM((P, D), jnp.float32),
        pltpu.SemaphoreType.DMA,
        pltpu.SemaphoreType.DMA,
        pltpu.SemaphoreType.DMA,
        pltpu.SemaphoreType.DMA,
        pltpu.SemaphoreType.DMA,
        pltpu.SemaphoreType.DMA,
    ],
)(_edge_body)


NBLK = 400
NGRID = N // NBLK
_ATOM_OFF = [0, 119, 123, 135, 147, 157, 163, 169, 171]
_ATOM_TOT = 256


def _encoder_body(x_ref, tabs_ref, out_ref):
    iota = lax.broadcasted_iota(jnp.int32, (NBLK, _ATOM_TOT), 1)
    oh = jnp.zeros((NBLK, _ATOM_TOT), jnp.float32)
    for i in range(len(ATOM_DIMS)):
        idx = x_ref[:, i:i + 1] + _ATOM_OFF[i]
        oh = oh + (iota == idx).astype(jnp.float32)
    out_ref[...] = jnp.dot(oh, tabs_ref[...],
                           preferred_element_type=jnp.float32, precision=lax.Precision.HIGHEST)


def _encoder(x, tabs):
    return pl.pallas_call(
        _encoder_body,
        grid=(NGRID,),
        in_specs=[pl.BlockSpec((NBLK, 9), lambda i: (i, 0)),
                  pl.BlockSpec((_ATOM_TOT, D), lambda i: (0, 0))],
        out_specs=pl.BlockSpec((NBLK, D), lambda i: (i, 0)),
        out_shape=jax.ShapeDtypeStruct((N, D), jnp.float32),
    )(x, tabs)


def _make_dense_body(has_res2):
    def body(mx_ref, mn_ref, sm_ref, cnt_ref, hres_ref, res2_ref,
             aW_ref, ab_ref, mW_ref, mb_ref, g_ref, b_ref,
             out_ref, hn_ref):
        cntv = cnt_ref[...]
        has = cntv > 0.0
        mx = jnp.where(has, mx_ref[...], 0.0)
        mn = jnp.where(has, mn_ref[...], 0.0)
        mean = jnp.where(has, sm_ref[...] / jnp.maximum(cntv, 1.0), 0.0)
        m = jnp.dot(mx, aW_ref[0], preferred_element_type=jnp.float32, precision=lax.Precision.HIGHEST)
        m = m + jnp.dot(mn, aW_ref[1], preferred_element_type=jnp.float32, precision=lax.Precision.HIGHEST)
        m = m + jnp.dot(mean, aW_ref[2], preferred_element_type=jnp.float32, precision=lax.Precision.HIGHEST)
        m = m + ab_ref[...]
        h = jnp.dot(hres_ref[...] + m, mW_ref[...],
                    preferred_element_type=jnp.float32, precision=lax.Precision.HIGHEST) + mb_ref[...]
        if has_res2:
            h = h + res2_ref[...]
        out_ref[...] = h
        mu = jnp.mean(h, axis=1, keepdims=True)
        var = jnp.mean((h - mu) ** 2, axis=1, keepdims=True)
        hn = (h - mu) / jnp.sqrt(var + 1e-5) * g_ref[...] + b_ref[...]
        hn_ref[...] = jnp.maximum(hn, 0.0)
    return body


def _dense_layer(mx, mn, sm, cnt, hres, res2, aW, ab, mW, mb, g, b, has_res2):
    return pl.pallas_call(
        _make_dense_body(has_res2),
        grid=(NGRID,),
        in_specs=[pl.BlockSpec((NBLK, D), lambda i: (i, 0)),
                  pl.BlockSpec((NBLK, D), lambda i: (i, 0)),
                  pl.BlockSpec((NBLK, D), lambda i: (i, 0)),
                  pl.BlockSpec((NBLK, 1), lambda i: (i, 0)),
                  pl.BlockSpec((NBLK, D), lambda i: (i, 0)),
                  pl.BlockSpec((NBLK, D), lambda i: (i, 0)),
                  pl.BlockSpec((3, D, D), lambda i: (0, 0, 0)),
                  pl.BlockSpec((1, D), lambda i: (0, 0)),
                  pl.BlockSpec((D, D), lambda i: (0, 0)),
                  pl.BlockSpec((1, D), lambda i: (0, 0)),
                  pl.BlockSpec((1, D), lambda i: (0, 0)),
                  pl.BlockSpec((1, D), lambda i: (0, 0))],
        out_specs=[pl.BlockSpec((NBLK, D), lambda i: (i, 0)),
                   pl.BlockSpec((NBLK, D), lambda i: (i, 0))],
        out_shape=[jax.ShapeDtypeStruct((N, D), jnp.float32),
                   jax.ShapeDtypeStruct((N, D), jnp.float32)],
    )(mx, mn, sm, cnt, hres, res2, aW, ab, mW, mb, g, b)


def _head_body(x_ref, hW_ref, hb_ref, g_ref, b_ref, eT_ref, bias_ref,
               out_ref):
    a = jnp.dot(x_ref[...], hW_ref[...],
                preferred_element_type=jnp.float32, precision=lax.Precision.HIGHEST) + hb_ref[...]
    a = jnp.maximum(a, 0.0)
    mu = jnp.mean(a, axis=1, keepdims=True)
    var = jnp.mean((a - mu) ** 2, axis=1, keepdims=True)
    a = (a - mu) / jnp.sqrt(var + 1e-5) * g_ref[...] + b_ref[...]
    out_ref[...] = jnp.dot(a, eT_ref[...],
                           preferred_element_type=jnp.float32, precision=lax.Precision.HIGHEST) + bias_ref[...]


def _head(x, hW, hb, g, b, eT, bias):
    return pl.pallas_call(
        _head_body,
        grid=(NGRID,),
        in_specs=[pl.BlockSpec((NBLK, D), lambda i: (i, 0)),
                  pl.BlockSpec((D, D), lambda i: (0, 0)),
                  pl.BlockSpec((1, D), lambda i: (0, 0)),
                  pl.BlockSpec((1, D), lambda i: (0, 0)),
                  pl.BlockSpec((1, D), lambda i: (0, 0)),
                  pl.BlockSpec((D, 128), lambda i: (0, 0)),
                  pl.BlockSpec((1, 128), lambda i: (0, 0))],
        out_specs=pl.BlockSpec((NBLK, 128), lambda i: (i, 0)),
        out_shape=jax.ShapeDtypeStruct((N, 128), jnp.float32),
    )(x, hW, hb, g, b, eT, bias)

def _ln(x, g, b, eps=1e-5):
    mu = jnp.mean(x, axis=-1, keepdims=True)
    var = jnp.var(x, axis=-1, keepdims=True)
    return (x - mu) / jnp.sqrt(var + eps) * g + b


def kernel(params, x, edge_index, edge_attr, batch):
    src = edge_index[0]
    dst = edge_index[1]
    cid = edge_attr[:, 0] * 12 + edge_attr[:, 1] * 2 + edge_attr[:, 2]

    pad = EPAD - E
    dst_p = jnp.concatenate([dst, jnp.full((pad,), NPAD, jnp.int32)])
    pay = jnp.concatenate([src * 64 + cid, jnp.zeros((pad,), jnp.int32)])
    dst_s, pay_s = jax.lax.sort((dst_p, pay), num_keys=1)
    src_s = lax.shift_right_logical(pay_s, 6)
    pk_s = dst_s * 64 + (pay_s & 63)

    bounds = jnp.searchsorted(
        dst_s, jnp.arange(0, NPAD + 1, P, dtype=jnp.int32)).astype(jnp.int32)
    bounds = jnp.concatenate([bounds, jnp.zeros((144 - NB - 1,), jnp.int32)])
    edges_at = jnp.searchsorted(
        dst_s, jnp.arange(N + 1, dtype=jnp.int32)).astype(jnp.int32)
    cnt = (edges_at[1:] - edges_at[:-1]).astype(jnp.float32)[:, None]

    bond = params['bond_emb']
    combos = []
    for l in range(L):
        t = (bond[0][l][:, None, None, :] + bond[1][l][None, :, None, :]
             + bond[2][l][None, None, :, :]).reshape(60, D)
        combos.append(jnp.concatenate([t, jnp.zeros((4, D), jnp.float32)], 0))

    tabs = jnp.zeros((_ATOM_TOT, D), jnp.float32)
    for i, off in enumerate(_ATOM_OFF):
        tabs = lax.dynamic_update_slice(tabs, params['atom_emb'][i], (off, 0))
    h = _encoder(x, tabs)

    def sc_conv(h_in, l):
        mxb, mnb, smi = _edge_kernel(h_in, src_s, pk_s, bounds, combos[l])
        return mxb[:N], mnb[:N], smi[:N]

    aW = params['aggr_W'].reshape(L, 3, D, D)
    zeros_res = jnp.zeros((N, D), jnp.float32)
    ln_g = params['ln_g']
    ln_b = params['ln_b']

    mx, mn, sm = sc_conv(h, 0)
    h1, hn = _dense_layer(mx, mn, sm, cnt, h, zeros_res, aW[0],
                          params['aggr_b'][0][None], params['mlp_W'][0],
                          params['mlp_b'][0][None], ln_g[1][None],
                          ln_b[1][None], has_res2=False)
    hcur, hncur = h1, hn
    for l in range(1, L):
        gi, bi = (l + 1, l + 1) if l < L - 1 else (0, 0)
        mx, mn, sm = sc_conv(hncur, l)
        hcur, hncur = _dense_layer(
            mx, mn, sm, cnt, hncur, hcur, aW[l],
            params['aggr_b'][l][None], params['mlp_W'][l],
            params['mlp_b'][l][None], ln_g[gi][None], ln_b[bi][None],
            has_res2=True)

    node_fea = hncur

    gcnt = jax.ops.segment_sum(jnp.ones((N,), jnp.float32), batch,
                               num_segments=G)[:, None]
    gmax = jnp.where(gcnt > 0,
                     jax.ops.segment_max(node_fea, batch, num_segments=G), 0.0)
    gmean = jax.ops.segment_sum(node_fea, batch, num_segments=G) / \
        jnp.maximum(gcnt, 1.0)
    graph_fea = jnp.concatenate([gmax, gmean], axis=-1)

    eT = jnp.zeros((D, 128), jnp.float32)
    eT = lax.dynamic_update_slice(eT, params['atom_emb'][0].T, (0, 0))
    bias = jnp.zeros((1, 128), jnp.float32)
    bias = lax.dynamic_update_slice(bias, params['head_bias'][None], (0, 0))
    atom_pred = _head(node_fea, params['head_W'], params['head_b'][None],
                      params['head_ln_g'][None], params['head_ln_b'][None],
                      eT, bias)[:, :119]
    return graph_fea, node_fea, atom_pred


# speedup vs baseline: 2.2223x; 2.2223x over previous
import functools

import jax
import jax.numpy as jnp
from jax import lax
from jax.experimental import pallas as pl
from jax.experimental.pallas import tpu as pltpu
from jax.experimental.pallas import tpu_sc as plsc

N = 10000
E = 160000
D = 256
L = 4
G = 64
ATOM_DIMS = [119, 4, 12, 12, 10, 6, 6, 2, 2]

NPAD = 10240
NB = 128
P = NPAD // NB
EPAD = E + 256
C = 96
NC = 2
NS = 16

_mesh = plsc.VectorSubcoreMesh(core_axis_name="c", subcore_axis_name="s")


def _edge_body(h_in, srcs, pks, bnds, comb,
               mx, mn, sm,
               bounds_v, combo_v, src_v0, src_v1, pk_v0, pk_v1,
               rows_v0, rows_v1, amx, amn, asm_,
               ssem0, ssem1, psem0, psem1, gsem0, gsem1):
    cidx = lax.axis_index("c")
    sidx = lax.axis_index("s")
    w = sidx * NC + cidx
    srcv = (src_v0, src_v1)
    pkv = (pk_v0, pk_v1)
    rowsv = (rows_v0, rows_v1)
    ssem = (ssem0, ssem1)
    psem = (psem0, psem1)
    gsem = (gsem0, gsem1)

    pltpu.sync_copy(bnds, bounds_v)
    pltpu.sync_copy(comb, combo_v)

    neg = jnp.full((16,), -jnp.inf, jnp.float32)
    pos = jnp.full((16,), jnp.inf, jnp.float32)
    zer = jnp.zeros((16,), jnp.float32)

    def bucket(bk, carry):
        b = w + 32 * bk
        node_base = b * P
        start = bounds_v[pl.ds(b, 16)][0]
        end = bounds_v[pl.ds(b + 1, 16)][0]
        start_a = start & jnp.int32(-8)
        nch = (end - start_a + (C - 1)) // C

        def initrow(r, cr):
            for jj in range(16):
                sl = pl.ds(16 * jj, 16)
                amx[r, sl] = neg
                amn[r, sl] = pos
                asm_[r, sl] = zer
            return cr
        lax.fori_loop(0, P, initrow, 0)

        def fire_meta(ci, k):
            base = pl.multiple_of(start_a + ci * C, 8)
            pltpu.async_copy(srcs.at[pl.ds(base, C)], srcv[k], ssem[k])
            pltpu.async_copy(pks.at[pl.ds(base, C)],
                             pkv[k].at[pl.ds(0, C)], psem[k])

        def wait_meta(k):
            pltpu.make_async_copy(srcs.at[pl.ds(0, C)], srcv[k],
                                  ssem[k]).wait()
            pltpu.make_async_copy(pks.at[pl.ds(0, C)],
                                  pkv[k].at[pl.ds(0, C)], psem[k]).wait()

        def fire_gather(k):
            pltpu.async_copy(h_in.at[srcv[k]], rowsv[k], gsem[k])

        def wait_gather(k):
            pltpu.make_async_copy(h_in.at[srcv[k]], rowsv[k], gsem[k]).wait()

        @pl.when(nch > 0)
        def _():
            fire_meta(0, 0)

            @pl.when(nch > 1)
            def _():
                fire_meta(1, 1)
            wait_meta(0)
            fire_gather(0)

        def step(ci, k):
            wait_gather(k)

            @pl.when(ci + 1 < nch)
            def _():
                wait_meta(1 - k)
                fire_gather(1 - k)

            def edge(i, ec, k=k):
                for u in range(2):
                    e = 2 * i + u
                    v = pkv[k][pl.ds(e, 16)][0]
                    off = lax.shift_right_logical(v, 6) - node_base
                    ck = v & 63

                    @pl.when((off >= 0) & (off < P))
                    def _(e=e, off=off, ck=ck, k=k):
                        for jj in range(16):
                            sl = pl.ds(16 * jj, 16)
                            m = rowsv[k][e, sl] + combo_v[ck, sl]
                            amx[off, sl] = jnp.maximum(amx[off, sl], m)
                            amn[off, sl] = jnp.minimum(amn[off, sl], m)
                            asm_[off, sl] = asm_[off, sl] + m
                return ec
            lax.fori_loop(0, C // 2, edge, 0)

            @pl.when(ci + 2 < nch)
            def _():
                fire_meta(ci + 2, k)

        def pair(i, cr):
            ci0 = 2 * i
            step(ci0, 0)

            @pl.when(ci0 + 1 < nch)
            def _():
                step(ci0 + 1, 1)
            return cr
        lax.fori_loop(0, (nch + 1) // 2, pair, 0)

        pltpu.sync_copy(amx, mx.at[pl.ds(node_base, P)])
        pltpu.sync_copy(amn, mn.at[pl.ds(node_base, P)])
        pltpu.sync_copy(asm_, sm.at[pl.ds(node_base, P)])
        return carry
    lax.fori_loop(0, 4, bucket, 0)


_edge_kernel = functools.partial(
    pl.kernel,
    out_type=[jax.ShapeDtypeStruct((NPAD, D), jnp.float32),
              jax.ShapeDtypeStruct((NPAD, D), jnp.float32),
              jax.ShapeDtypeStruct((NPAD, D), jnp.float32)],
    mesh=_mesh,
    scratch_types=[
        pltpu.VMEM((144,), jnp.int32),
        pltpu.VMEM((64, D), jnp.float32),
        pltpu.VMEM((C,), jnp.int32),
        pltpu.VMEM((C,), jnp.int32),
        pltpu.VMEM((C + 16,), jnp.int32),
        pltpu.VMEM((C + 16,), jnp.int32),
        pltpu.VMEM((C, D), jnp.float32),
        pltpu.VMEM((C, D), jnp.float32),
        pltpu.VMEM((P, D), jnp.float32),
        pltpu.VMEM((P, D), jnp.float32),
        pltpu.VMEM((P, D), jnp.float32),
        pltpu.SemaphoreType.DMA,
        pltpu.SemaphoreType.DMA,
        pltpu.SemaphoreType.DMA,
        pltpu.SemaphoreType.DMA,
        pltpu.SemaphoreType.DMA,
        pltpu.SemaphoreType.DMA,
    ],
)(_edge_body)


NBLK = 400
NGRID = N // NBLK
_ATOM_OFF = [0, 119, 123, 135, 147, 157, 163, 169, 171]
_ATOM_TOT = 256


def _encoder_body(x_ref, tabs_ref, out_ref):
    iota = lax.broadcasted_iota(jnp.int32, (NBLK, _ATOM_TOT), 1)
    oh = jnp.zeros((NBLK, _ATOM_TOT), jnp.float32)
    for i in range(len(ATOM_DIMS)):
        idx = x_ref[:, i:i + 1] + _ATOM_OFF[i]
        oh = oh + (iota == idx).astype(jnp.float32)
    out_ref[...] = jnp.dot(oh, tabs_ref[...],
                           preferred_element_type=jnp.float32, precision=lax.Precision.HIGHEST)


def _encoder(x, tabs):
    return pl.pallas_call(
        _encoder_body,
        grid=(NGRID,),
        in_specs=[pl.BlockSpec((NBLK, 9), lambda i: (i, 0)),
                  pl.BlockSpec((_ATOM_TOT, D), lambda i: (0, 0))],
        out_specs=pl.BlockSpec((NBLK, D), lambda i: (i, 0)),
        out_shape=jax.ShapeDtypeStruct((N, D), jnp.float32),
    )(x, tabs)


def _make_dense_body(has_res2):
    def body(mx_ref, mn_ref, sm_ref, cnt_ref, hres_ref, res2_ref,
             aW_ref, ab_ref, mW_ref, mb_ref, g_ref, b_ref,
             out_ref, hn_ref):
        cntv = cnt_ref[...]
        has = cntv > 0.0
        mx = jnp.where(has, mx_ref[...], 0.0)
        mn = jnp.where(has, mn_ref[...], 0.0)
        mean = jnp.where(has, sm_ref[...] / jnp.maximum(cntv, 1.0), 0.0)
        m = jnp.dot(mx, aW_ref[0], preferred_element_type=jnp.float32, precision=lax.Precision.HIGHEST)
        m = m + jnp.dot(mn, aW_ref[1], preferred_element_type=jnp.float32, precision=lax.Precision.HIGHEST)
        m = m + jnp.dot(mean, aW_ref[2], preferred_element_type=jnp.float32, precision=lax.Precision.HIGHEST)
        m = m + ab_ref[...]
        h = jnp.dot(hres_ref[...] + m, mW_ref[...],
                    preferred_element_type=jnp.float32, precision=lax.Precision.HIGHEST) + mb_ref[...]
        if has_res2:
            h = h + res2_ref[...]
        out_ref[...] = h
        mu = jnp.mean(h, axis=1, keepdims=True)
        var = jnp.mean((h - mu) ** 2, axis=1, keepdims=True)
        hn = (h - mu) / jnp.sqrt(var + 1e-5) * g_ref[...] + b_ref[...]
        hn_ref[...] = jnp.maximum(hn, 0.0)
    return body


def _dense_layer(mx, mn, sm, cnt, hres, res2, aW, ab, mW, mb, g, b, has_res2):
    return pl.pallas_call(
        _make_dense_body(has_res2),
        grid=(NGRID,),
        in_specs=[pl.BlockSpec((NBLK, D), lambda i: (i, 0)),
                  pl.BlockSpec((NBLK, D), lambda i: (i, 0)),
                  pl.BlockSpec((NBLK, D), lambda i: (i, 0)),
                  pl.BlockSpec((NBLK, 1), lambda i: (i, 0)),
                  pl.BlockSpec((NBLK, D), lambda i: (i, 0)),
                  pl.BlockSpec((NBLK, D), lambda i: (i, 0)),
                  pl.BlockSpec((3, D, D), lambda i: (0, 0, 0)),
                  pl.BlockSpec((1, D), lambda i: (0, 0)),
                  pl.BlockSpec((D, D), lambda i: (0, 0)),
                  pl.BlockSpec((1, D), lambda i: (0, 0)),
                  pl.BlockSpec((1, D), lambda i: (0, 0)),
                  pl.BlockSpec((1, D), lambda i: (0, 0))],
        out_specs=[pl.BlockSpec((NBLK, D), lambda i: (i, 0)),
                   pl.BlockSpec((NBLK, D), lambda i: (i, 0))],
        out_shape=[jax.ShapeDtypeStruct((N, D), jnp.float32),
                   jax.ShapeDtypeStruct((N, D), jnp.float32)],
    )(mx, mn, sm, cnt, hres, res2, aW, ab, mW, mb, g, b)


def _head_body(x_ref, hW_ref, hb_ref, g_ref, b_ref, eT_ref, bias_ref,
               out_ref):
    a = jnp.dot(x_ref[...], hW_ref[...],
                preferred_element_type=jnp.float32, precision=lax.Precision.HIGHEST) + hb_ref[...]
    a = jnp.maximum(a, 0.0)
    mu = jnp.mean(a, axis=1, keepdims=True)
    var = jnp.mean((a - mu) ** 2, axis=1, keepdims=True)
    a = (a - mu) / jnp.sqrt(var + 1e-5) * g_ref[...] + b_ref[...]
    out_ref[...] = jnp.dot(a, eT_ref[...],
                           preferred_element_type=jnp.float32, precision=lax.Precision.HIGHEST) + bias_ref[...]


def _head(x, hW, hb, g, b, eT, bias):
    return pl.pallas_call(
        _head_body,
        grid=(NGRID,),
        in_specs=[pl.BlockSpec((NBLK, D), lambda i: (i, 0)),
                  pl.BlockSpec((D, D), lambda i: (0, 0)),
                  pl.BlockSpec((1, D), lambda i: (0, 0)),
                  pl.BlockSpec((1, D), lambda i: (0, 0)),
                  pl.BlockSpec((1, D), lambda i: (0, 0)),
                  pl.BlockSpec((D, 128), lambda i: (0, 0)),
                  pl.BlockSpec((1, 128), lambda i: (0, 0))],
        out_specs=pl.BlockSpec((NBLK, 128), lambda i: (i, 0)),
        out_shape=jax.ShapeDtypeStruct((N, 128), jnp.float32),
    )(x, hW, hb, g, b, eT, bias)

def _ln(x, g, b, eps=1e-5):
    mu = jnp.mean(x, axis=-1, keepdims=True)
    var = jnp.var(x, axis=-1, keepdims=True)
    return (x - mu) / jnp.sqrt(var + eps) * g + b


def kernel(params, x, edge_index, edge_attr, batch):
    src = edge_index[0]
    dst = edge_index[1]
    cid = edge_attr[:, 0] * 12 + edge_attr[:, 1] * 2 + edge_attr[:, 2]

    pad = EPAD - E
    dst_p = jnp.concatenate([dst, jnp.full((pad,), NPAD, jnp.int32)])
    pay = jnp.concatenate([src * 64 + cid, jnp.zeros((pad,), jnp.int32)])
    dst_s, pay_s = jax.lax.sort((dst_p, pay), num_keys=1)
    src_s = lax.shift_right_logical(pay_s, 6)
    pk_s = dst_s * 64 + (pay_s & 63)

    bounds = jnp.searchsorted(
        dst_s, jnp.arange(0, NPAD + 1, P, dtype=jnp.int32)).astype(jnp.int32)
    bounds = jnp.concatenate([bounds, jnp.zeros((144 - NB - 1,), jnp.int32)])
    edges_at = jnp.searchsorted(
        dst_s, jnp.arange(N + 1, dtype=jnp.int32)).astype(jnp.int32)
    cnt = (edges_at[1:] - edges_at[:-1]).astype(jnp.float32)[:, None]

    bond = params['bond_emb']
    combos = []
    for l in range(L):
        t = (bond[0][l][:, None, None, :] + bond[1][l][None, :, None, :]
             + bond[2][l][None, None, :, :]).reshape(60, D)
        combos.append(jnp.concatenate([t, jnp.zeros((4, D), jnp.float32)], 0))

    tabs = jnp.zeros((_ATOM_TOT, D), jnp.float32)
    for i, off in enumerate(_ATOM_OFF):
        tabs = lax.dynamic_update_slice(tabs, params['atom_emb'][i], (off, 0))
    h = _encoder(x, tabs)

    def sc_conv(h_in, l):
        mxb, mnb, smi = _edge_kernel(h_in, src_s, pk_s, bounds, combos[l])
        return mxb[:N], mnb[:N], smi[:N]

    aW = params['aggr_W'].reshape(L, 3, D, D)
    zeros_res = jnp.zeros((N, D), jnp.float32)
    ln_g = params['ln_g']
    ln_b = params['ln_b']

    mx, mn, sm = sc_conv(h, 0)
    h1, hn = _dense_layer(mx, mn, sm, cnt, h, zeros_res, aW[0],
                          params['aggr_b'][0][None], params['mlp_W'][0],
                          params['mlp_b'][0][None], ln_g[1][None],
                          ln_b[1][None], has_res2=False)
    hcur, hncur = h1, hn
    for l in range(1, L):
        gi, bi = (l + 1, l + 1) if l < L - 1 else (0, 0)
        mx, mn, sm = sc_conv(hncur, l)
        hcur, hncur = _dense_layer(
            mx, mn, sm, cnt, hncur, hcur, aW[l],
            params['aggr_b'][l][None], params['mlp_W'][l],
            params['mlp_b'][l][None], ln_g[gi][None], ln_b[bi][None],
            has_res2=True)

    node_fea = hncur

    gcnt = jax.ops.segment_sum(jnp.ones((N,), jnp.float32), batch,
                               num_segments=G)[:, None]
    gmax = jnp.where(gcnt > 0,
                     jax.ops.segment_max(node_fea, batch, num_segments=G), 0.0)
    gmean = jax.ops.segment_sum(node_fea, batch, num_segments=G) / \
        jnp.maximum(gcnt, 1.0)
    graph_fea = jnp.concatenate([gmax, gmean], axis=-1)

    eT = jnp.zeros((D, 128), jnp.float32)
    eT = lax.dynamic_update_slice(eT, params['atom_emb'][0].T, (0, 0))
    bias = jnp.zeros((1, 128), jnp.float32)
    bias = lax.dynamic_update_slice(bias, params['head_bias'][None], (0, 0))
    atom_pred = _head(node_fea, params['head_W'], params['head_b'][None],
                      params['head_ln_g'][None], params['head_ln_b'][None],
                      eT, bias)[:, :119]
    return graph_fea, node_fea, atom_pred
